# baseline (speedup 1.0000x reference)
import functools

import jax
import jax.numpy as jnp
from jax import lax
from jax.experimental import pallas as pl
from jax.experimental.pallas import tpu as pltpu
from jax.experimental.pallas import tpu_sc as plsc

NXY = 512
NPIX = NXY * NXY
N_POINTS = 500000

GROUPS = 8
QUARTS = 4
QSIZE = NPIX // QUARTS
VBITS = 11
BLK = 3968
BLKS = 16
CHUNK = BLK * BLKS
N_PAD = CHUNK * GROUPS
PAD = N_PAD - N_POINTS
UNROLL = 8

_mesh = plsc.VectorSubcoreMesh(core_axis_name="c", subcore_axis_name="s")


@functools.partial(
    pl.kernel,
    out_type=jax.ShapeDtypeStruct((GROUPS, QUARTS, QSIZE), jnp.int32),
    mesh=_mesh,
    compiler_params=pltpu.CompilerParams(needs_layout_passes=False),
    scratch_types=[
        pltpu.VMEM((QSIZE,), jnp.int32),
        pltpu.VMEM((BLK,), jnp.int32),
        pltpu.VMEM((BLK,), jnp.int32),
        pltpu.SemaphoreType.DMA,
        pltpu.SemaphoreType.DMA,
    ],
)
def _scatter_max_kernel(w_hbm, part_hbm, canvas, bufa, bufb, sem0, sem1):
    cid = lax.axis_index("c")
    sid = lax.axis_index("s")
    wid = cid * 16 + sid
    g = wid // QUARTS
    q = wid % QUARTS
    lo2 = q * (QSIZE << VBITS)
    hi2 = lo2 + (QSIZE << VBITS)

    bufs = (bufa, bufb)
    sems = (sem0, sem1)

    def _start(blk, buf):
        base = pl.multiple_of(g * CHUNK + blk * BLK, 16)
        return (pltpu.async_copy(w_hbm.at[pl.ds(base, BLK)], bufs[buf],
                                 sems[buf]),)

    pend = _start(0, 0)

    zeros16 = jnp.zeros((16,), jnp.int32)

    def _zero(i, carry):
        o = pl.multiple_of(i * 128, 128)
        for u in range(8):
            canvas[pl.ds(o + u * 16, 16)] = zeros16
        return carry

    lax.fori_loop(0, QSIZE // 128, _zero, 0)

    def _decode(tb, o):
        w = tb[pl.ds(o, 16)]
        msk = (w >= lo2) & (w < hi2)
        li = jnp.where(msk, (w - lo2) >> VBITS, 0)
        v = jnp.where(msk, w & ((1 << VBITS) - 1), 0)
        return li, v

    for blk in range(BLKS):
        buf = blk & 1
        for h in pend:
            h.wait()
        if blk + 1 < BLKS:
            pend = _start(blk + 1, (blk + 1) & 1)

        tb = bufs[buf]

        def _grp(t, carry, tb=tb):
            o0 = pl.multiple_of(t * (16 * UNROLL), 16 * UNROLL)
            decs = [_decode(tb, pl.multiple_of(o0 + u * 16, 16))
                    for u in range(UNROLL)]
            for li, v in decs:
                old = plsc.load_gather(canvas, [li])
                plsc.store_scatter(canvas, [li], jnp.maximum(old, v))
            m = None
            for li, v in decs:
                need = plsc.load_gather(canvas, [li]) < v
                m = need if m is None else m | need

            def _fc(mm):
                return jnp.any(mm)

            def _fb(mm):
                nm = None
                for u in range(UNROLL):
                    li, v = _decode(tb, pl.multiple_of(o0 + u * 16, 16))
                    cur = plsc.load_gather(canvas, [li])
                    plsc.store_scatter(canvas, [li], jnp.maximum(cur, v),
                                       mask=cur < v)
                    cur2 = plsc.load_gather(canvas, [li])
                    bad = cur2 < v
                    nm = bad if nm is None else nm | bad
                return nm

            lax.while_loop(_fc, _fb, m)
            return carry

        lax.fori_loop(0, BLK // (16 * UNROLL), _grp, 0)

    pltpu.sync_copy(canvas, part_hbm.at[g, q])


def _merge_body(p_ref, o_ref):
    m = jnp.max(p_ref[...], axis=0)
    o_ref[...] = (m & 1).astype(jnp.float32)


_merge = pl.pallas_call(
    _merge_body,
    grid=(64,),
    in_specs=[pl.BlockSpec((GROUPS, 8, NXY), lambda i: (0, i, 0))],
    out_specs=pl.BlockSpec((8, NXY), lambda i: (i, 0)),
    out_shape=jax.ShapeDtypeStruct((NXY, NXY), jnp.float32),
)


def kernel(points, voxel_coords):
    c = voxel_coords.astype(jnp.int32)
    bit = (jnp.abs(points[:, 4]) > 0.1).astype(jnp.int32)
    w = ((c[:, 1] * NXY + c[:, 2]) << VBITS) | (2 * c[:, 0] + 2 + bit)
    w = jnp.pad(w, (0, PAD), constant_values=NPIX << VBITS)
    parts = _scatter_max_kernel(w)
    img = _merge(parts.reshape(GROUPS, NXY, NXY))
    return img.reshape(1, 1, NXY, NXY)


# speedup vs baseline: 1.0826x; 1.0826x over previous
import functools

import jax
import jax.numpy as jnp
from jax import lax
from jax.experimental import pallas as pl
from jax.experimental.pallas import tpu as pltpu
from jax.experimental.pallas import tpu_sc as plsc

NXY = 512
NPIX = NXY * NXY
N_POINTS = 500000

GROUPS = 8
QUARTS = 4
QSIZE = NPIX // QUARTS
VBITS = 11
BLK = 3968
BLKS = 16
CHUNK = BLK * BLKS
N_PAD = CHUNK * GROUPS
PAD = N_PAD - N_POINTS
UNROLL = 8

_mesh = plsc.VectorSubcoreMesh(core_axis_name="c", subcore_axis_name="s")


@functools.partial(
    pl.kernel,
    out_type=jax.ShapeDtypeStruct((GROUPS, QUARTS, QSIZE), jnp.int32),
    mesh=_mesh,
    compiler_params=pltpu.CompilerParams(needs_layout_passes=False),
    scratch_types=[
        pltpu.VMEM((QSIZE,), jnp.int32),
        pltpu.VMEM((BLK,), jnp.int32),
        pltpu.VMEM((BLK,), jnp.int32),
        pltpu.SemaphoreType.DMA,
        pltpu.SemaphoreType.DMA,
    ],
)
def _scatter_max_kernel(w_hbm, part_hbm, canvas, bufa, bufb, sem0, sem1):
    cid = lax.axis_index("c")
    sid = lax.axis_index("s")
    wid = cid * 16 + sid
    g = wid // QUARTS
    q = wid % QUARTS
    lo2 = q * (QSIZE << VBITS)
    hi2 = lo2 + (QSIZE << VBITS)

    bufs = (bufa, bufb)
    sems = (sem0, sem1)

    def _start(blk, buf):
        base = pl.multiple_of(g * CHUNK + blk * BLK, 16)
        return (pltpu.async_copy(w_hbm.at[pl.ds(base, BLK)], bufs[buf],
                                 sems[buf]),)

    pend = _start(0, 0)

    zeros16 = jnp.zeros((16,), jnp.int32)

    def _zero(i, carry):
        o = pl.multiple_of(i * 128, 128)
        for u in range(8):
            canvas[pl.ds(o + u * 16, 16)] = zeros16
        return carry

    lax.fori_loop(0, QSIZE // 128, _zero, 0)

    def _decode(tb, o):
        w = tb[pl.ds(o, 16)]
        msk = (w >= lo2) & (w < hi2)
        li = jnp.where(msk, (w - lo2) >> VBITS, 0)
        v = jnp.where(msk, w & ((1 << VBITS) - 1), 0)
        return li, v

    for blk in range(BLKS):
        buf = blk & 1
        for h in pend:
            h.wait()
        if blk + 1 < BLKS:
            pend = _start(blk + 1, (blk + 1) & 1)

        tb = bufs[buf]

        def _grp(t, carry, tb=tb):
            o0 = pl.multiple_of(t * (16 * UNROLL), 16 * UNROLL)
            decs = [_decode(tb, pl.multiple_of(o0 + u * 16, 16))
                    for u in range(UNROLL)]
            for a in range(0, UNROLL, 4):
                quad = decs[a:a + 4]
                olds = [plsc.load_gather(canvas, [li]) for li, _ in quad]
                for (li, v), old in zip(quad, olds):
                    plsc.store_scatter(canvas, [li], jnp.maximum(old, v))
            m = None
            for li, v in decs:
                need = plsc.load_gather(canvas, [li]) < v
                m = need if m is None else m | need

            def _fc(mm):
                return jnp.any(mm)

            def _fb(mm):
                nm = None
                for u in range(UNROLL):
                    li, v = _decode(tb, pl.multiple_of(o0 + u * 16, 16))
                    cur = plsc.load_gather(canvas, [li])
                    plsc.store_scatter(canvas, [li], jnp.maximum(cur, v),
                                       mask=cur < v)
                    cur2 = plsc.load_gather(canvas, [li])
                    bad = cur2 < v
                    nm = bad if nm is None else nm | bad
                return nm

            lax.while_loop(_fc, _fb, m)
            return carry

        lax.fori_loop(0, BLK // (16 * UNROLL), _grp, 0)

    pltpu.sync_copy(canvas, part_hbm.at[g, q])


def _merge_body(p_ref, o_ref):
    m = jnp.max(p_ref[...], axis=0)
    o_ref[...] = (m & 1).astype(jnp.float32)


_merge = pl.pallas_call(
    _merge_body,
    grid=(64,),
    in_specs=[pl.BlockSpec((GROUPS, 8, NXY), lambda i: (0, i, 0))],
    out_specs=pl.BlockSpec((8, NXY), lambda i: (i, 0)),
    out_shape=jax.ShapeDtypeStruct((NXY, NXY), jnp.float32),
)


def kernel(points, voxel_coords):
    c = voxel_coords.astype(jnp.int32)
    bit = (jnp.abs(points[:, 4]) > 0.1).astype(jnp.int32)
    w = ((c[:, 1] * NXY + c[:, 2]) << VBITS) | (2 * c[:, 0] + 2 + bit)
    w = jnp.pad(w, (0, PAD), constant_values=NPIX << VBITS)
    parts = _scatter_max_kernel(w)
    img = _merge(parts.reshape(GROUPS, NXY, NXY))
    return img.reshape(1, 1, NXY, NXY)


# speedup vs baseline: 1.0920x; 1.0087x over previous
import functools

import jax
import jax.numpy as jnp
from jax import lax
from jax.experimental import pallas as pl
from jax.experimental.pallas import tpu as pltpu
from jax.experimental.pallas import tpu_sc as plsc

NXY = 512
NPIX = NXY * NXY
N_POINTS = 500000

GROUPS = 8
QUARTS = 4
QSIZE = NPIX // QUARTS
VBITS = 11
BLK = 3968
BLKS = 16
CHUNK = BLK * BLKS
N_PAD = CHUNK * GROUPS
PAD = N_PAD - N_POINTS
UNROLL = 8

_mesh = plsc.VectorSubcoreMesh(core_axis_name="c", subcore_axis_name="s")


@functools.partial(
    pl.kernel,
    out_type=jax.ShapeDtypeStruct((GROUPS, QUARTS, QSIZE), jnp.int32),
    mesh=_mesh,
    compiler_params=pltpu.CompilerParams(needs_layout_passes=False),
    scratch_types=[
        pltpu.VMEM((QSIZE,), jnp.int32),
        pltpu.VMEM((BLK,), jnp.int32),
        pltpu.VMEM((BLK,), jnp.int32),
        pltpu.SemaphoreType.DMA,
        pltpu.SemaphoreType.DMA,
    ],
)
def _scatter_max_kernel(w_hbm, part_hbm, canvas, bufa, bufb, sem0, sem1):
    cid = lax.axis_index("c")
    sid = lax.axis_index("s")
    wid = cid * 16 + sid
    g = wid // QUARTS
    q = wid % QUARTS
    lo2 = q * (QSIZE << VBITS)
    hi2 = lo2 + (QSIZE << VBITS)

    bufs = (bufa, bufb)
    sems = (sem0, sem1)

    def _start(blk, buf):
        base = pl.multiple_of(g * CHUNK + blk * BLK, 16)
        return (pltpu.async_copy(w_hbm.at[pl.ds(base, BLK)], bufs[buf],
                                 sems[buf]),)

    pend = _start(0, 0)

    zeros16 = jnp.zeros((16,), jnp.int32)

    def _zero(i, carry):
        o = pl.multiple_of(i * 128, 128)
        for u in range(8):
            canvas[pl.ds(o + u * 16, 16)] = zeros16
        return carry

    lax.fori_loop(0, QSIZE // 128, _zero, 0)

    def _decode(tb, o):
        w = tb[pl.ds(o, 16)]
        msk = (w >= lo2) & (w < hi2)
        li = jnp.where(msk, (w - lo2) >> VBITS, 0)
        v = jnp.where(msk, w & ((1 << VBITS) - 1), 0)
        return li, v

    for blk in range(BLKS):
        buf = blk & 1
        for h in pend:
            h.wait()
        if blk + 1 < BLKS:
            pend = _start(blk + 1, (blk + 1) & 1)

        tb = bufs[buf]

        def _grp(t, carry, tb=tb):
            o0 = pl.multiple_of(t * (16 * UNROLL), 16 * UNROLL)
            decs = [_decode(tb, pl.multiple_of(o0 + u * 16, 16))
                    for u in range(UNROLL)]
            olds = [plsc.load_gather(canvas, [li]) for li, _ in decs]
            for (li, v), old in zip(decs, olds):
                plsc.store_scatter(canvas, [li], jnp.maximum(old, v))
            m = None
            for li, v in decs:
                need = plsc.load_gather(canvas, [li]) < v
                m = need if m is None else m | need

            def _fc(mm):
                return jnp.any(mm)

            def _fb(mm):
                nm = None
                for u in range(UNROLL):
                    li, v = _decode(tb, pl.multiple_of(o0 + u * 16, 16))
                    cur = plsc.load_gather(canvas, [li])
                    plsc.store_scatter(canvas, [li], jnp.maximum(cur, v),
                                       mask=cur < v)
                    cur2 = plsc.load_gather(canvas, [li])
                    bad = cur2 < v
                    nm = bad if nm is None else nm | bad
                return nm

            lax.while_loop(_fc, _fb, m)
            return carry

        lax.fori_loop(0, BLK // (16 * UNROLL), _grp, 0)

    pltpu.sync_copy(canvas, part_hbm.at[g, q])


def _merge_body(p_ref, o_ref):
    m = jnp.max(p_ref[...], axis=0)
    o_ref[...] = (m & 1).astype(jnp.float32)


_merge = pl.pallas_call(
    _merge_body,
    grid=(64,),
    in_specs=[pl.BlockSpec((GROUPS, 8, NXY), lambda i: (0, i, 0))],
    out_specs=pl.BlockSpec((8, NXY), lambda i: (i, 0)),
    out_shape=jax.ShapeDtypeStruct((NXY, NXY), jnp.float32),
)


def kernel(points, voxel_coords):
    c = voxel_coords.astype(jnp.int32)
    bit = (jnp.abs(points[:, 4]) > 0.1).astype(jnp.int32)
    w = ((c[:, 1] * NXY + c[:, 2]) << VBITS) | (2 * c[:, 0] + 2 + bit)
    w = jnp.pad(w, (0, PAD), constant_values=NPIX << VBITS)
    parts = _scatter_max_kernel(w)
    img = _merge(parts.reshape(GROUPS, NXY, NXY))
    return img.reshape(1, 1, NXY, NXY)


# speedup vs baseline: 1.3722x; 1.2566x over previous
import functools

import jax
import jax.numpy as jnp
from jax import lax
from jax.experimental import pallas as pl
from jax.experimental.pallas import tpu as pltpu
from jax.experimental.pallas import tpu_sc as plsc

NXY = 512
NPIX = NXY * NXY
N_POINTS = 500000

GROUPS = 8
QUARTS = 4
QSIZE = NPIX // QUARTS
VBITS = 11
BLK = 3968
BLKS = 16
CHUNK = BLK * BLKS
N_PAD = CHUNK * GROUPS
PAD = N_PAD - N_POINTS
UNROLL = 8

_mesh = plsc.VectorSubcoreMesh(core_axis_name="c", subcore_axis_name="s")


@functools.partial(
    pl.kernel,
    out_type=jax.ShapeDtypeStruct((GROUPS, QUARTS, QSIZE), jnp.int32),
    mesh=_mesh,
    compiler_params=pltpu.CompilerParams(needs_layout_passes=False),
    scratch_types=[
        pltpu.VMEM((QSIZE,), jnp.int32),
        pltpu.VMEM((BLK,), jnp.int32),
        pltpu.VMEM((BLK,), jnp.int32),
        pltpu.SemaphoreType.DMA,
        pltpu.SemaphoreType.DMA,
    ],
)
def _scatter_max_kernel(w_hbm, part_hbm, canvas, bufa, bufb, sem0, sem1):
    cid = lax.axis_index("c")
    sid = lax.axis_index("s")
    wid = cid * 16 + sid
    g = wid // QUARTS
    q = wid % QUARTS
    lo2 = q * (QSIZE << VBITS)
    hi2 = lo2 + (QSIZE << VBITS)

    bufs = (bufa, bufb)
    sems = (sem0, sem1)

    def _start(blk, buf):
        base = pl.multiple_of(g * CHUNK + blk * BLK, 16)
        return (pltpu.async_copy(w_hbm.at[pl.ds(base, BLK)], bufs[buf],
                                 sems[buf]),)

    pend = _start(0, 0)

    zeros16 = jnp.zeros((16,), jnp.int32)

    def _zero(i, carry):
        o = pl.multiple_of(i * 128, 128)
        for u in range(8):
            canvas[pl.ds(o + u * 16, 16)] = zeros16
        return carry

    lax.fori_loop(0, QSIZE // 128, _zero, 0)

    def _decode(tb, o):
        w = tb[pl.ds(o, 16)]
        msk = (w >= lo2) & (w < hi2)
        li = jnp.where(msk, (w - lo2) >> VBITS, 0)
        v = jnp.where(msk, w & ((1 << VBITS) - 1), 0)
        return li, v

    for blk in range(BLKS):
        buf = blk & 1
        for h in pend:
            h.wait()
        if blk + 1 < BLKS:
            pend = _start(blk + 1, (blk + 1) & 1)

        tb = bufs[buf]

        def _grp(t, carry, tb=tb):
            o0 = pl.multiple_of(t * (16 * UNROLL), 16 * UNROLL)
            decs = [_decode(tb, pl.multiple_of(o0 + u * 16, 16))
                    for u in range(UNROLL)]
            olds = [plsc.load_gather(canvas, [li]) for li, _ in decs]
            for (li, v), old in zip(decs, olds):
                plsc.store_scatter(canvas, [li], jnp.maximum(old, v))
            m = None
            for li, v in decs:
                need = plsc.load_gather(canvas, [li]) < v
                m = need if m is None else m | need

            def _fc(mm):
                return jnp.any(mm)

            def _fb(mm):
                nm = None
                for u in range(UNROLL):
                    li, v = _decode(tb, pl.multiple_of(o0 + u * 16, 16))
                    cur = plsc.load_gather(canvas, [li])
                    plsc.store_scatter(canvas, [li], jnp.maximum(cur, v),
                                       mask=cur < v)
                    cur2 = plsc.load_gather(canvas, [li])
                    bad = cur2 < v
                    nm = bad if nm is None else nm | bad
                return nm

            lax.while_loop(_fc, _fb, m)
            return carry

        lax.fori_loop(0, BLK // (16 * UNROLL), _grp, 0)

    pltpu.sync_copy(canvas, part_hbm.at[g, q])


def _merge_body(p_ref, o_ref):
    m = jnp.max(p_ref[...], axis=0)
    o_ref[...] = (m & 1).astype(jnp.float32)


_merge = pl.pallas_call(
    _merge_body,
    grid=(8,),
    in_specs=[pl.BlockSpec((GROUPS, 64, NXY), lambda i: (0, i, 0))],
    out_specs=pl.BlockSpec((64, NXY), lambda i: (i, 0)),
    out_shape=jax.ShapeDtypeStruct((NXY, NXY), jnp.float32),
)


def kernel(points, voxel_coords):
    c = voxel_coords.astype(jnp.int32)
    bit = (jnp.abs(points[:, 4]) > 0.1).astype(jnp.int32)
    w = ((c[:, 1] * NXY + c[:, 2]) << VBITS) | (2 * c[:, 0] + 2 + bit)
    w = jnp.pad(w, (0, PAD), constant_values=NPIX << VBITS)
    parts = _scatter_max_kernel(w)
    img = _merge(parts.reshape(GROUPS, NXY, NXY))
    return img.reshape(1, 1, NXY, NXY)


# speedup vs baseline: 1.4051x; 1.0240x over previous
import functools

import jax
import jax.numpy as jnp
from jax import lax
from jax.experimental import pallas as pl
from jax.experimental.pallas import tpu as pltpu
from jax.experimental.pallas import tpu_sc as plsc

NXY = 512
NPIX = NXY * NXY
N_POINTS = 500000

GROUPS = 8
QUARTS = 4
QSIZE = NPIX // QUARTS
VBITS = 11
BLK = 3968
BLKS = 16
CHUNK = BLK * BLKS
N_PAD = CHUNK * GROUPS
PAD = N_PAD - N_POINTS
UNROLL = 8

_mesh = plsc.VectorSubcoreMesh(core_axis_name="c", subcore_axis_name="s")


@functools.partial(
    pl.kernel,
    out_type=jax.ShapeDtypeStruct((GROUPS, QUARTS, QSIZE), jnp.int32),
    mesh=_mesh,
    compiler_params=pltpu.CompilerParams(needs_layout_passes=False),
    scratch_types=[
        pltpu.VMEM((QSIZE,), jnp.int32),
        pltpu.VMEM((BLK,), jnp.int32),
        pltpu.VMEM((BLK,), jnp.int32),
        pltpu.SemaphoreType.DMA,
        pltpu.SemaphoreType.DMA,
    ],
)
def _scatter_max_kernel(w_hbm, part_hbm, canvas, bufa, bufb, sem0, sem1):
    cid = lax.axis_index("c")
    sid = lax.axis_index("s")
    wid = cid * 16 + sid
    g = wid // QUARTS
    q = wid % QUARTS
    lo2 = q * (QSIZE << VBITS)
    hi2 = lo2 + (QSIZE << VBITS)

    bufs = (bufa, bufb)
    sems = (sem0, sem1)

    def _start(blk, buf):
        base = pl.multiple_of(g * CHUNK + blk * BLK, 16)
        return (pltpu.async_copy(w_hbm.at[pl.ds(base, BLK)], bufs[buf],
                                 sems[buf]),)

    pend = _start(0, 0)

    zeros16 = jnp.zeros((16,), jnp.int32)

    def _zero(i, carry):
        o = pl.multiple_of(i * 128, 128)
        for u in range(8):
            canvas[pl.ds(o + u * 16, 16)] = zeros16
        return carry

    lax.fori_loop(0, QSIZE // 128, _zero, 0)

    def _decode(tb, o):
        w = tb[pl.ds(o, 16)]
        msk = (w >= lo2) & (w < hi2)
        li = jnp.where(msk, (w - lo2) >> VBITS, 0)
        v = jnp.where(msk, w & ((1 << VBITS) - 1), 0)
        return li, v

    for blk in range(BLKS):
        buf = blk & 1
        for h in pend:
            h.wait()
        if blk + 1 < BLKS:
            pend = _start(blk + 1, (blk + 1) & 1)

        tb = bufs[buf]

        def _grp(t, carry, tb=tb):
            o0 = pl.multiple_of(t * (16 * UNROLL), 16 * UNROLL)
            decs = [_decode(tb, pl.multiple_of(o0 + u * 16, 16))
                    for u in range(UNROLL)]
            olds = [plsc.load_gather(canvas, [li]) for li, _ in decs]
            for (li, v), old in zip(decs, olds):
                plsc.store_scatter(canvas, [li], jnp.maximum(old, v))
            m = None
            for li, v in decs:
                need = plsc.load_gather(canvas, [li]) < v
                m = need if m is None else m | need

            def _fc(mm):
                return jnp.any(mm)

            def _fb(mm):
                nm = None
                for u in range(UNROLL):
                    li, v = _decode(tb, pl.multiple_of(o0 + u * 16, 16))
                    cur = plsc.load_gather(canvas, [li])
                    plsc.store_scatter(canvas, [li], jnp.maximum(cur, v),
                                       mask=cur < v)
                    cur2 = plsc.load_gather(canvas, [li])
                    bad = cur2 < v
                    nm = bad if nm is None else nm | bad
                return nm

            lax.while_loop(_fc, _fb, m)
            return carry

        lax.fori_loop(0, BLK // (16 * UNROLL), _grp, 0)

    pltpu.sync_copy(canvas, part_hbm.at[g, q])


def _merge_body(p_ref, o_ref):
    m = jnp.max(p_ref[...], axis=0)
    o_ref[...] = (m & 1).astype(jnp.float32)


_merge = pl.pallas_call(
    _merge_body,
    grid=(2,),
    in_specs=[pl.BlockSpec((GROUPS, 256, NXY), lambda i: (0, i, 0))],
    out_specs=pl.BlockSpec((256, NXY), lambda i: (i, 0)),
    out_shape=jax.ShapeDtypeStruct((NXY, NXY), jnp.float32),
)


def kernel(points, voxel_coords):
    c = voxel_coords.astype(jnp.int32)
    bit = (jnp.abs(points[:, 4]) > 0.1).astype(jnp.int32)
    w = ((c[:, 1] * NXY + c[:, 2]) << VBITS) | (2 * c[:, 0] + 2 + bit)
    w = jnp.pad(w, (0, PAD), constant_values=NPIX << VBITS)
    parts = _scatter_max_kernel(w)
    img = _merge(parts.reshape(GROUPS, NXY, NXY))
    return img.reshape(1, 1, NXY, NXY)
